# baseline (speedup 1.0000x reference)
import jax
import jax.numpy as jnp
from jax.experimental import pallas as pl
from jax.experimental.pallas import tpu as pltpu

HIDDEN = 128


def _round_up(n, m):
    return ((n + m - 1) // m) * m


def _make_body(seq_len, block_t, n_t, b_blk, unroll=8):
    H = HIDDEN
    rem = seq_len - (n_t - 1) * block_t
    hb = b_blk // 2

    def body(x_ref,
             w_ih0_ref,
             w_step_ref,
             b1_ref,
             w_d1_ref, b_d1_ref,
             w_d2_ref, b_d2_ref,
             w_head_ref, b_head_ref,
             out_ref,
             h0_ref, c0_ref, h1_ref, c1_ref,
             rec0_ref,
             g0_ref):
        t_blk = pl.program_id(1)

        @pl.when(t_blk == 0)
        def _init():
            z = jnp.zeros((b_blk, H), jnp.float32)
            h0_ref[...] = z
            c0_ref[...] = z
            h1_ref[...] = z
            c1_ref[...] = z
            rec0_ref[...] = jnp.zeros((b_blk, 4 * H), jnp.float32)

        cw = x_ref.shape[-1]
        g0_ref[...] = jnp.dot(x_ref[...].reshape(block_t * b_blk, cw),
                              w_ih0_ref[...],
                              preferred_element_type=jnp.float32)

        w_step = w_step_ref[...]
        b1b = jnp.broadcast_to(b1_ref[...], (hb, 4 * H))

        def gates0(g0, c0):
            s = jax.nn.sigmoid(g0[:, :3 * H])
            gg = jnp.tanh(g0[:, 3 * H:])
            cn = s[:, H:2 * H] * c0 + s[:, :H] * gg
            hn = s[:, 2 * H:] * jnp.tanh(cn)
            return hn, cn

        def lstm_block(n_steps):
            def step(t, carry):
                (h0a, c0a, h1a, c1a, r0a,
                 h0b, c0b, h1b, c1b, r0b) = carry
                row = pl.multiple_of(t * b_blk, b_blk)

                g0a = g0_ref[pl.ds(row, hb), :] + r0a
                h0na, c0na = gates0(g0a, c0a)
                lhsa = jnp.concatenate([h0na, h1a], axis=-1).astype(w_step.dtype)
                biga = jnp.dot(lhsa, w_step, preferred_element_type=jnp.float32)

                g0b = g0_ref[pl.ds(row + hb, hb), :] + r0b
                h0nb, c0nb = gates0(g0b, c0b)
                lhsb = jnp.concatenate([h0nb, h1b], axis=-1).astype(w_step.dtype)
                bigb = jnp.dot(lhsb, w_step, preferred_element_type=jnp.float32)

                g1a = biga[:, :4 * H] + b1b
                r0na = biga[:, 4 * H:]
                h1na, c1na = gates0(g1a, c1a)

                g1b = bigb[:, :4 * H] + b1b
                r0nb = bigb[:, 4 * H:]
                h1nb, c1nb = gates0(g1b, c1b)

                return (h0na, c0na, h1na, c1na, r0na,
                        h0nb, c0nb, h1nb, c1nb, r0nb)

            carry = (h0_ref[:hb], c0_ref[:hb], h1_ref[:hb], c1_ref[:hb],
                     rec0_ref[:hb],
                     h0_ref[hb:], c0_ref[hb:], h1_ref[hb:], c1_ref[hb:],
                     rec0_ref[hb:])
            (h0a, c0a, h1a, c1a, r0a,
             h0b, c0b, h1b, c1b, r0b) = jax.lax.fori_loop(
                0, n_steps, step, carry, unroll=min(unroll, n_steps))
            h0_ref[:hb] = h0a
            c0_ref[:hb] = c0a
            h1_ref[:hb] = h1a
            c1_ref[:hb] = c1a
            rec0_ref[:hb] = r0a
            h0_ref[hb:] = h0b
            c0_ref[hb:] = c0b
            h1_ref[hb:] = h1b
            c1_ref[hb:] = c1b
            rec0_ref[hb:] = r0b

        if rem == block_t:
            lstm_block(block_t)
        else:
            @pl.when(t_blk < n_t - 1)
            def _full():
                lstm_block(block_t)

            @pl.when(t_blk == n_t - 1)
            def _tail():
                lstm_block(rem)

        @pl.when(t_blk == n_t - 1)
        def _epilogue():
            h = jnp.maximum(
                jnp.dot(h1_ref[...], w_d1_ref[...],
                        preferred_element_type=jnp.float32) + b_d1_ref[...],
                0.0)
            h = jnp.maximum(
                jnp.dot(h, w_d2_ref[...],
                        preferred_element_type=jnp.float32) + b_d2_ref[...],
                0.0)
            out_ref[0] = (
                jnp.dot(h, w_head_ref[...], preferred_element_type=jnp.float32)
                + b_head_ref[...])

    return body


def kernel(beatmap_features, positions, w_ih0, w_step, b1, w_d1, b_d1,
           w_d2, b_d2, w_head, b_head, *, block_t=128):
    H = HIDDEN
    x = jnp.concatenate([beatmap_features, positions],
                        axis=-1).astype(jnp.float32)
    B, T, C = x.shape

    b_pad = _round_up(max(B, 1), 8)
    n_b = 2 if b_pad % 32 == 0 else 1
    b_blk = b_pad // n_b

    bt = _round_up(max(8, min(block_t, _round_up(T, 8))), 8)
    T_pad = _round_up(T, bt)
    n_t = T_pad // bt

    x = jnp.transpose(x, (1, 0, 2))
    x = jnp.pad(x, ((0, T_pad - T), (0, b_pad - B), (0, 0)))
    x = jnp.concatenate(
        [x, jnp.ones((T_pad, b_pad, 1), jnp.float32)], axis=-1)

    L2 = w_head.shape[1]
    const = lambda b, t: (0, 0)

    out = pl.pallas_call(
        _make_body(seq_len=T, block_t=bt, n_t=n_t, b_blk=b_blk),
        grid=(n_b, n_t),
        in_specs=[
            pl.BlockSpec((bt, b_blk, C + 1), lambda b, t: (t, b, 0)),
            pl.BlockSpec(w_ih0.shape, const),
            pl.BlockSpec(w_step.shape, const),
            pl.BlockSpec(b1.shape, const),
            pl.BlockSpec(w_d1.shape, const),
            pl.BlockSpec(b_d1.shape, const),
            pl.BlockSpec(w_d2.shape, const),
            pl.BlockSpec(b_d2.shape, const),
            pl.BlockSpec(w_head.shape, const),
            pl.BlockSpec(b_head.shape, const),
        ],
        out_specs=pl.BlockSpec((1, b_blk, L2), lambda b, t: (b, 0, 0)),
        out_shape=jax.ShapeDtypeStruct((n_b, b_blk, L2), jnp.float32),
        scratch_shapes=[
            pltpu.VMEM((b_blk, H), jnp.float32),
            pltpu.VMEM((b_blk, H), jnp.float32),
            pltpu.VMEM((b_blk, H), jnp.float32),
            pltpu.VMEM((b_blk, H), jnp.float32),
            pltpu.VMEM((b_blk, 4 * H), jnp.float32),
            pltpu.VMEM((bt * b_blk, 4 * H), jnp.float32),
        ],
        compiler_params=pltpu.CompilerParams(
            dimension_semantics=("parallel", "arbitrary"),
        ),
    )(x, w_ih0, w_step, b1, w_d1, b_d1, w_d2, b_d2, w_head, b_head)

    out = out.reshape(n_b * b_blk, L2)
    L = L2 // 2
    return out[:B, :L], out[:B, L:]


# speedup vs baseline: 1.0391x; 1.0391x over previous
import jax
import jax.numpy as jnp
from jax.experimental import pallas as pl
from jax.experimental.pallas import tpu as pltpu

HIDDEN = 128
PROJ_CHUNK = 256


def _round_up(n, m):
    return ((n + m - 1) // m) * m


def _sigmoid(x):
    return jax.nn.sigmoid(x)


def _make_body(seq_len, block_t, n_t, b_blk, c_in, unroll=8):
    H = HIDDEN
    rem = seq_len - (n_t - 1) * block_t
    hb = b_blk // 2
    n_chunks = (block_t * b_blk) // PROJ_CHUNK
    t_per_chunk = PROJ_CHUNK // b_blk

    def body(x_ref,
             w_ih0_ref,
             w_step_ref,
             b1_ref,
             w_d1_ref, b_d1_ref,
             w_d2_ref, b_d2_ref,
             w_head_ref, b_head_ref,
             out_ref,
             h0_ref, c0_ref, h1_ref, c1_ref,
             rec0_ref,
             g0_ref):
        t_blk = pl.program_id(1)

        @pl.when(t_blk == 0)
        def _init():
            z = jnp.zeros((b_blk, H), jnp.float32)
            h0_ref[...] = z
            c0_ref[...] = z
            h1_ref[...] = z
            c1_ref[...] = z
            rec0_ref[...] = jnp.zeros((b_blk, 4 * H), jnp.float32)
            for mxu in (0, 1):
                for base in (0, 32):
                    pltpu.matmul_pop(acc_addr=base, shape=(hb, 256),
                                     dtype=jnp.float32, mxu_index=mxu)
                    pltpu.matmul_pop(acc_addr=base + 8, shape=(hb, 256),
                                     dtype=jnp.float32, mxu_index=mxu)
                for base in (64, 128):
                    pltpu.matmul_pop(acc_addr=base, shape=(PROJ_CHUNK, 256),
                                     dtype=jnp.float32, mxu_index=mxu)

        wzr = jnp.zeros((256 - c_in, 256), jnp.bfloat16)
        w0p = jnp.concatenate(
            [w_ih0_ref[...][:, :256].astype(jnp.bfloat16), wzr], axis=0)
        w1p = jnp.concatenate(
            [w_ih0_ref[...][:, 256:].astype(jnp.bfloat16), wzr], axis=0)
        pltpu.matmul_push_rhs(w0p, staging_register=0, mxu_index=0)
        pltpu.matmul_push_rhs(w1p, staging_register=0, mxu_index=1)
        zpad = jnp.zeros((PROJ_CHUNK, 256 - c_in), jnp.bfloat16)
        for k in range(n_chunks):
            xc = x_ref[pl.ds(k * t_per_chunk, t_per_chunk)]
            xc = xc.reshape(PROJ_CHUNK, c_in)
            xhi = xc.astype(jnp.bfloat16)
            xlo = (xc - xhi.astype(jnp.float32)).astype(jnp.bfloat16)
            xhi = jnp.concatenate([xhi, zpad], axis=1)
            xlo = jnp.concatenate([xlo, zpad], axis=1)
            base = 64 if (k % 2 == 0) else 128
            lsr0 = 0 if k == 0 else None
            pltpu.matmul_acc_lhs(acc_addr=base, lhs=xhi, mxu_index=0,
                                 load_staged_rhs=lsr0)
            pltpu.matmul_acc_lhs(acc_addr=base, lhs=xlo, mxu_index=0,
                                 load_staged_rhs=None)
            pltpu.matmul_acc_lhs(acc_addr=base, lhs=xhi, mxu_index=1,
                                 load_staged_rhs=lsr0)
            pltpu.matmul_acc_lhs(acc_addr=base, lhs=xlo, mxu_index=1,
                                 load_staged_rhs=None)
            gl = pltpu.matmul_pop(acc_addr=base, shape=(PROJ_CHUNK, 256),
                                  dtype=jnp.float32, mxu_index=0)
            gr = pltpu.matmul_pop(acc_addr=base, shape=(PROJ_CHUNK, 256),
                                  dtype=jnp.float32, mxu_index=1)
            g0_ref[pl.ds(k * PROJ_CHUNK, PROJ_CHUNK), :] = jnp.concatenate(
                [gl, gr], axis=1)

        b1b = jnp.broadcast_to(b1_ref[...], (hb, 4 * H))

        def step_matmul(h0n, h1, base):
            lhs = jnp.concatenate([h0n, h1], axis=-1).astype(jnp.bfloat16)
            pltpu.matmul_push_rhs(w_step_ref[:, 0:256],
                                  staging_register=0, mxu_index=0)
            pltpu.matmul_push_rhs(w_step_ref[:, 256:512],
                                  staging_register=1, mxu_index=0)
            pltpu.matmul_push_rhs(w_step_ref[:, 512:768],
                                  staging_register=0, mxu_index=1)
            pltpu.matmul_push_rhs(w_step_ref[:, 768:1024],
                                  staging_register=1, mxu_index=1)
            pltpu.matmul_acc_lhs(acc_addr=base, lhs=lhs, mxu_index=0,
                                 load_staged_rhs=0)
            pltpu.matmul_acc_lhs(acc_addr=base + 8, lhs=lhs, mxu_index=0,
                                 load_staged_rhs=1)
            pltpu.matmul_acc_lhs(acc_addr=base, lhs=lhs, mxu_index=1,
                                 load_staged_rhs=0)
            pltpu.matmul_acc_lhs(acc_addr=base + 8, lhs=lhs, mxu_index=1,
                                 load_staged_rhs=1)

        def step_pop(base):
            g1l = pltpu.matmul_pop(acc_addr=base, shape=(hb, 256),
                                   dtype=jnp.float32, mxu_index=0)
            g1r = pltpu.matmul_pop(acc_addr=base + 8, shape=(hb, 256),
                                   dtype=jnp.float32, mxu_index=0)
            r0l = pltpu.matmul_pop(acc_addr=base, shape=(hb, 256),
                                   dtype=jnp.float32, mxu_index=1)
            r0r = pltpu.matmul_pop(acc_addr=base + 8, shape=(hb, 256),
                                   dtype=jnp.float32, mxu_index=1)
            g1 = jnp.concatenate([g1l, g1r], axis=1) + b1b
            rec0n = jnp.concatenate([r0l, r0r], axis=1)
            return g1, rec0n

        def gates(g, c):
            s = _sigmoid(g[:, :3 * H])
            gg = jnp.tanh(g[:, 3 * H:])
            cn = s[:, H:2 * H] * c + s[:, :H] * gg
            hn = s[:, 2 * H:] * jnp.tanh(cn)
            return hn, cn

        def lstm_block(n_steps):
            def step(t, carry):
                (h0a, c0a, h1a, c1a, r0a,
                 h0b, c0b, h1b, c1b, r0b) = carry
                row = pl.multiple_of(t * b_blk, b_blk)

                g0a = g0_ref[pl.ds(row, hb), :] + r0a
                h0na, c0na = gates(g0a, c0a)
                step_matmul(h0na, h1a, 0)

                g0b = g0_ref[pl.ds(row + hb, hb), :] + r0b
                h0nb, c0nb = gates(g0b, c0b)
                step_matmul(h0nb, h1b, 32)

                g1a, r0na = step_pop(0)
                h1na, c1na = gates(g1a, c1a)

                g1b, r0nb = step_pop(32)
                h1nb, c1nb = gates(g1b, c1b)

                return (h0na, c0na, h1na, c1na, r0na,
                        h0nb, c0nb, h1nb, c1nb, r0nb)

            carry = (h0_ref[:hb], c0_ref[:hb], h1_ref[:hb], c1_ref[:hb],
                     rec0_ref[:hb],
                     h0_ref[hb:], c0_ref[hb:], h1_ref[hb:], c1_ref[hb:],
                     rec0_ref[hb:])
            (h0a, c0a, h1a, c1a, r0a,
             h0b, c0b, h1b, c1b, r0b) = jax.lax.fori_loop(
                0, n_steps, step, carry, unroll=min(unroll, n_steps))
            h0_ref[:hb] = h0a
            c0_ref[:hb] = c0a
            h1_ref[:hb] = h1a
            c1_ref[:hb] = c1a
            rec0_ref[:hb] = r0a
            h0_ref[hb:] = h0b
            c0_ref[hb:] = c0b
            h1_ref[hb:] = h1b
            c1_ref[hb:] = c1b
            rec0_ref[hb:] = r0b

        if rem == block_t:
            lstm_block(block_t)
        else:
            @pl.when(t_blk < n_t - 1)
            def _full():
                lstm_block(block_t)

            @pl.when(t_blk == n_t - 1)
            def _tail():
                lstm_block(rem)

        @pl.when(t_blk == n_t - 1)
        def _epilogue():
            def small_matmul(lhs, w, kdim):
                n = w.shape[1]
                wp = jnp.concatenate(
                    [w, jnp.zeros((kdim, 256 - n), jnp.float32)], axis=1)
                wp = jnp.concatenate(
                    [wp, jnp.zeros((256 - kdim, 256), jnp.float32)], axis=0)
                whi = wp.astype(jnp.bfloat16)
                wlo = (wp - whi.astype(jnp.float32)).astype(jnp.bfloat16)
                lp = jnp.concatenate(
                    [lhs, jnp.zeros((b_blk, 256 - kdim), jnp.float32)],
                    axis=1)
                lhi = lp.astype(jnp.bfloat16)
                llo = (lp - lhi.astype(jnp.float32)).astype(jnp.bfloat16)
                pltpu.matmul_push_rhs(whi, staging_register=0, mxu_index=0)
                pltpu.matmul_push_rhs(wlo, staging_register=1, mxu_index=0)
                pltpu.matmul_acc_lhs(acc_addr=64, lhs=lhi, mxu_index=0,
                                     load_staged_rhs=0)
                pltpu.matmul_acc_lhs(acc_addr=64, lhs=llo, mxu_index=0,
                                     load_staged_rhs=None)
                pltpu.matmul_acc_lhs(acc_addr=64 + 16, lhs=lhi, mxu_index=0,
                                     load_staged_rhs=1)
                r = pltpu.matmul_pop(acc_addr=64, shape=(b_blk, 256),
                                     dtype=jnp.float32, mxu_index=0)
                r2 = pltpu.matmul_pop(acc_addr=64 + 16, shape=(b_blk, 256),
                                      dtype=jnp.float32, mxu_index=0)
                return (r + r2)[:, :n]

            h = jnp.maximum(
                small_matmul(h1_ref[...], w_d1_ref[...], H)
                + b_d1_ref[...], 0.0)
            h = jnp.maximum(
                small_matmul(h, w_d2_ref[...], H) + b_d2_ref[...], 0.0)
            out_ref[0] = (small_matmul(h, w_head_ref[...], w_d2_ref.shape[1])
                          + b_head_ref[...])

    return body


def kernel(beatmap_features, positions, w_ih0, w_step, b1, w_d1, b_d1,
           w_d2, b_d2, w_head, b_head, *, block_t=128):
    H = HIDDEN
    x = jnp.concatenate([beatmap_features, positions],
                        axis=-1).astype(jnp.float32)
    B, T, C = x.shape

    b_pad = _round_up(max(B, 1), 8)
    n_b = 2 if b_pad % 32 == 0 else 1
    b_blk = b_pad // n_b

    bt = _round_up(max(8, min(block_t, _round_up(T, 8))), 8)
    T_pad = _round_up(T, bt)
    n_t = T_pad // bt

    x = jnp.transpose(x, (1, 0, 2))
    x = jnp.pad(x, ((0, T_pad - T), (0, b_pad - B), (0, 0)))
    x = jnp.concatenate(
        [x, jnp.ones((T_pad, b_pad, 1), jnp.float32)], axis=-1)

    L2 = w_head.shape[1]
    const = lambda b, t: (0, 0)

    out = pl.pallas_call(
        _make_body(seq_len=T, block_t=bt, n_t=n_t, b_blk=b_blk, c_in=C + 1),
        grid=(n_b, n_t),
        in_specs=[
            pl.BlockSpec((bt, b_blk, C + 1), lambda b, t: (t, b, 0)),
            pl.BlockSpec(w_ih0.shape, const),
            pl.BlockSpec(w_step.shape, const),
            pl.BlockSpec(b1.shape, const),
            pl.BlockSpec(w_d1.shape, const),
            pl.BlockSpec(b_d1.shape, const),
            pl.BlockSpec(w_d2.shape, const),
            pl.BlockSpec(b_d2.shape, const),
            pl.BlockSpec(w_head.shape, const),
            pl.BlockSpec(b_head.shape, const),
        ],
        out_specs=pl.BlockSpec((1, b_blk, L2), lambda b, t: (b, 0, 0)),
        out_shape=jax.ShapeDtypeStruct((n_b, b_blk, L2), jnp.float32),
        scratch_shapes=[
            pltpu.VMEM((b_blk, H), jnp.float32),
            pltpu.VMEM((b_blk, H), jnp.float32),
            pltpu.VMEM((b_blk, H), jnp.float32),
            pltpu.VMEM((b_blk, H), jnp.float32),
            pltpu.VMEM((b_blk, 4 * H), jnp.float32),
            pltpu.VMEM((bt * b_blk, 4 * H), jnp.float32),
        ],
        compiler_params=pltpu.CompilerParams(
            dimension_semantics=("parallel", "arbitrary"),
        ),
    )(x, w_ih0, w_step, b1, w_d1, b_d1, w_d2, b_d2, w_head, b_head)

    out = out.reshape(n_b * b_blk, L2)
    L = L2 // 2
    return out[:B, :L], out[:B, L:]
